# baseline (speedup 1.0000x reference)
import functools
import math

import jax
import jax.numpy as jnp
from jax.experimental import pallas as pl
from jax.experimental.pallas import tpu as pltpu

KEEP_RATIO = 0.5
MIN_TOKENS = 1


def _score_body(aw_ref, scores_ref, acc_ref):
    h = pl.program_id(1)
    acc_ref[h, :] = jnp.sum(aw_ref[0, 0], axis=0)

    @pl.when(h == pl.num_programs(1) - 1)
    def _():
        scores_ref[0, 0, :] = jnp.mean(acc_ref[...], axis=0)


def _keep_mask(s, k):
    T = s.shape[0]
    u = jax.lax.bitcast_convert_type(s, jnp.int32)

    def bs_body(i, p):
        cand = p | (jnp.int32(1) << (jnp.int32(30) - i))
        cnt = jnp.sum(jnp.where(u >= cand, 1, 0))
        return jnp.where(cnt >= k, cand, p)

    vstar = jax.lax.fori_loop(0, 31, bs_body, jnp.int32(0))
    gt = u > vstar
    eq = u == vstar
    m = k - jnp.sum(jnp.where(gt, 1, 0))
    pos = jax.lax.broadcasted_iota(jnp.int32, (T, 1), 0)[:, 0]
    nbits = max(1, (T - 1).bit_length() + 1)

    def ix_body(i, c):
        cand = c | (jnp.int32(1) << (jnp.int32(nbits - 1) - i))
        cnt = jnp.sum(jnp.where(eq & (pos < cand), 1, 0))
        return jnp.where(cnt < m, cand, c)

    c0 = jax.lax.fori_loop(0, nbits, ix_body, jnp.int32(0))
    keep_ties = eq & (pos <= c0) & (m > 0)
    return gt | keep_ties | (pos == 0)


def _prune_body(k, scores_ref, hs_ref, out_ref, mask_ref):
    s = scores_ref[0, 0, :]
    keep = _keep_mask(s, k)
    mask_ref[0, 0, :] = keep.astype(jnp.int32)
    out_ref[0] = hs_ref[0] * keep.astype(out_ref.dtype)[:, None]


@jax.jit
def kernel(hidden_states, attention_weights):
    B, T, D = hidden_states.shape
    _, H, _, _ = attention_weights.shape
    k = min(max(MIN_TOKENS, math.ceil(KEEP_RATIO * T)), T)

    scores = pl.pallas_call(
        _score_body,
        grid=(B, H),
        in_specs=[pl.BlockSpec((1, 1, T, T), lambda b, h: (b, h, 0, 0))],
        out_specs=pl.BlockSpec((1, 1, T), lambda b, h: (b, 0, 0)),
        out_shape=jax.ShapeDtypeStruct((B, 1, T), jnp.float32),
        scratch_shapes=[pltpu.VMEM((H, T), jnp.float32)],
        compiler_params=pltpu.CompilerParams(
            dimension_semantics=("arbitrary", "arbitrary"),
        ),
    )(attention_weights)

    pruned, mask_i32 = pl.pallas_call(
        functools.partial(_prune_body, k),
        grid=(B,),
        in_specs=[
            pl.BlockSpec((1, 1, T), lambda b: (b, 0, 0)),
            pl.BlockSpec((1, T, D), lambda b: (b, 0, 0)),
        ],
        out_specs=[
            pl.BlockSpec((1, T, D), lambda b: (b, 0, 0)),
            pl.BlockSpec((1, 1, T), lambda b: (b, 0, 0)),
        ],
        out_shape=[
            jax.ShapeDtypeStruct((B, T, D), hidden_states.dtype),
            jax.ShapeDtypeStruct((B, 1, T), jnp.int32),
        ],
    )(scores, hidden_states)

    return (pruned, mask_i32.reshape(B, T).astype(bool))
